# baseline (speedup 1.0000x reference)
import functools

import jax
import jax.numpy as jnp
from jax import lax
from jax.experimental import pallas as pl
from jax.experimental.pallas import tpu as pltpu
from jax.experimental.pallas import tpu_sc as plsc

BATCH = 16384
HIST = 50
DIM = 32
NUM_WORKERS = 32
BW = BATCH // NUM_WORKERS
LANES = 16

_mesh = plsc.VectorSubcoreMesh(core_axis_name="c", subcore_axis_name="s")


@functools.partial(
    pl.kernel,
    mesh=_mesh,
    out_type=jax.ShapeDtypeStruct((HIST, 4, 8 * BATCH), jnp.float32),
    scratch_types=[
        pltpu.VMEM((2, BW), jnp.int32),
        pltpu.VMEM((2, 4, 128, DIM), jnp.float32),
        pltpu.VMEM((2, 4 * 4096), jnp.float32),
        pltpu.SemaphoreType.DMA,
        pltpu.SemaphoreType.DMA,
        pltpu.SemaphoreType.DMA,
        pltpu.SemaphoreType.DMA,
    ],
    compiler_params=pltpu.CompilerParams(
        use_tc_tiling_on_sc=False,
        needs_layout_passes=False,
        disable_bounds_checks=True,
    ),
)
def _sc_gather(xt_hbm, table_hbm, out_hbm, idx_v, rows_v, tile_v, g0, g1, w0, w1):
    gsem = (g0, g1)
    wsem = (w0, w1)
    wid = lax.axis_index("s") * 2 + lax.axis_index("c")
    b0 = wid * BW

    lane = lax.iota(jnp.int32, LANES)

    def load_and_fire(h, b):
        pltpu.sync_copy(xt_hbm.at[h, pl.ds(b0, BW)], idx_v.at[b])
        for cc in range(4):
            pltpu.async_copy(
                table_hbm.at[idx_v.at[b, pl.ds(cc * 128, 128)]],
                rows_v.at[b, cc],
                gsem[b],
            )

    def wait_gather(b):
        for cc in range(4):
            pltpu.make_async_copy(
                table_hbm.at[pl.ds(0, 128)], rows_v.at[b, cc], gsem[b]
            ).wait()

    def wait_writeback(b):
        for r in range(4):
            pltpu.make_async_copy(
                out_hbm.at[0, 0, pl.ds(0, 4096)],
                tile_v.at[b, pl.ds(r * 4096, 4096)],
                wsem[b],
            ).wait()

    load_and_fire(0, 0)
    load_and_fire(1, 1)

    def pair_body(g, carry):
        for b in range(2):
            h = 2 * g + b
            wait_gather(b)

            @pl.when(g >= 1)
            def _():
                wait_writeback(b)

            def p_body(p, carry2):
                bc_vec = lane + 16 * p
                off = 16 * p
                for cc in range(4):
                    src = rows_v.at[b, cc]
                    for r in range(4):
                        for dr in range(8):
                            dvec = jnp.full((LANES,), 8 * r + dr, jnp.int32)
                            v = plsc.load_gather(src, [bc_vec, dvec])
                            dst0 = r * 4096 + cc * 1024 + dr * 128
                            tile_v[b, pl.ds(dst0 + off, LANES)] = v
                return carry2

            lax.fori_loop(0, 8, p_body, 0)

            for r in range(4):
                pltpu.async_copy(
                    tile_v.at[b, pl.ds(r * 4096, 4096)],
                    out_hbm.at[h, r, pl.ds(wid * 4096, 4096)],
                    wsem[b],
                )

            @pl.when(h + 2 < HIST)
            def _():
                load_and_fire(h + 2, b)

        return carry

    lax.fori_loop(0, HIST // 2, pair_body, 0)
    wait_writeback(0)
    wait_writeback(1)


def kernel(x, table):
    xt = x.T.astype(jnp.int32)
    out6 = _sc_gather(xt, table).reshape(HIST, 4, 128, 8, 128)
    return out6.transpose(2, 4, 0, 1, 3).reshape(BATCH, HIST, DIM)


# speedup vs baseline: 1.1249x; 1.1249x over previous
import functools

import jax
import jax.numpy as jnp
from jax import lax
from jax.experimental import pallas as pl
from jax.experimental.pallas import tpu as pltpu
from jax.experimental.pallas import tpu_sc as plsc

BATCH = 16384
HIST = 50
DIM = 32
NUM_WORKERS = 32
BW = BATCH // NUM_WORKERS
LANES = 16

_mesh = plsc.VectorSubcoreMesh(core_axis_name="c", subcore_axis_name="s")


@functools.partial(
    pl.kernel,
    mesh=_mesh,
    out_type=jax.ShapeDtypeStruct((HIST, 4, 8 * BATCH), jnp.float32),
    scratch_types=[
        pltpu.VMEM((2, BW), jnp.int32),
        pltpu.VMEM((2, 4, 128, DIM), jnp.float32),
        pltpu.VMEM((2, 4 * 4096), jnp.float32),
        pltpu.SemaphoreType.DMA,
        pltpu.SemaphoreType.DMA,
        pltpu.SemaphoreType.DMA,
        pltpu.SemaphoreType.DMA,
    ],
    compiler_params=pltpu.CompilerParams(
        use_tc_tiling_on_sc=False,
        needs_layout_passes=False,
        disable_bounds_checks=True,
    ),
)
def _sc_gather(xt_hbm, table_hbm, out_hbm, idx_v, rows_v, tile_v, g0, g1, w0, w1):
    gsem = (g0, g1)
    wsem = (w0, w1)
    wid = lax.axis_index("s") * 2 + lax.axis_index("c")
    b0 = wid * BW

    lane = lax.iota(jnp.int32, LANES)
    perm0 = ((lane >> 3) << 12) + ((lane & 7) << 7)
    perm1 = perm0 + 8192

    def load_and_fire(h, b):
        pltpu.sync_copy(xt_hbm.at[h, pl.ds(b0, BW)], idx_v.at[b])
        for cc in range(4):
            pltpu.async_copy(
                table_hbm.at[idx_v.at[b, pl.ds(cc * 128, 128)]],
                rows_v.at[b, cc],
                gsem[b],
            )

    def wait_gather(b):
        for cc in range(4):
            pltpu.make_async_copy(
                table_hbm.at[pl.ds(0, 128)], rows_v.at[b, cc], gsem[b]
            ).wait()

    def wait_writeback(b):
        for r in range(4):
            pltpu.make_async_copy(
                out_hbm.at[0, 0, pl.ds(0, 4096)],
                tile_v.at[b, pl.ds(r * 4096, 4096)],
                wsem[b],
            ).wait()

    load_and_fire(0, 0)
    load_and_fire(1, 1)

    def pair_body(g, carry):
        for b in range(2):
            h = 2 * g + b
            wait_gather(b)

            @pl.when(g >= 1)
            def _():
                wait_writeback(b)

            def bc_body(it, carry2):
                for k in range(4):
                    bc = 4 * it + k
                    for cc in range(4):
                        base = cc * 1024 + bc
                        v0 = rows_v[b, cc, bc, pl.ds(0, LANES)]
                        v1 = rows_v[b, cc, bc, pl.ds(LANES, LANES)]
                        plsc.store_scatter(tile_v.at[b], [perm0 + base], v0)
                        plsc.store_scatter(tile_v.at[b], [perm1 + base], v1)
                return carry2

            lax.fori_loop(0, 32, bc_body, 0)

            for r in range(4):
                pltpu.async_copy(
                    tile_v.at[b, pl.ds(r * 4096, 4096)],
                    out_hbm.at[h, r, pl.ds(wid * 4096, 4096)],
                    wsem[b],
                )

            @pl.when(h + 2 < HIST)
            def _():
                load_and_fire(h + 2, b)

        return carry

    lax.fori_loop(0, HIST // 2, pair_body, 0)
    wait_writeback(0)
    wait_writeback(1)


def kernel(x, table):
    xt = x.T.astype(jnp.int32)
    out6 = _sc_gather(xt, table).reshape(HIST, 4, 128, 8, 128)
    return out6.transpose(2, 4, 0, 1, 3).reshape(BATCH, HIST, DIM)
